# Initial kernel scaffold; baseline (speedup 1.0000x reference)
import jax
import jax.numpy as jnp
from jax.experimental import pallas as pl


def kernel(sketch_query_vectors, ref_key_vectors, G):
    raise NotImplementedError("write your pallas kernel here")


# speedup vs baseline: 37.5501x; 37.5501x over previous
import jax
import jax.numpy as jnp
from jax.experimental import pallas as pl

_EPS = 1e-8
_MARGIN = 0.6
_C = 256
_F = 32
_HW = _F * _F
_IMG = 256


def _triplet_kernel(sq_ref, rk_ref, gx_ref, gy_ref, out_ref):
    num = jnp.float32(0.0)
    den = jnp.float32(0.0)
    col = jax.lax.broadcasted_iota(jnp.int32, (_HW, _HW), 1)
    row = jax.lax.broadcasted_iota(jnp.int32, (_HW, _HW), 0)
    for b in range(sq_ref.shape[0]):
        xq = sq_ref[b]
        xr = rk_ref[b]
        qn = jnp.maximum(jnp.sqrt(jnp.sum(xq * xq, axis=0, keepdims=True)), _EPS)
        rn = jnp.maximum(jnp.sqrt(jnp.sum(xr * xr, axis=0, keepdims=True)), _EPS)
        xqn = xq / qn
        xrn = xr / rn
        sims = jax.lax.dot_general(
            xqn, xrn, (((0,), (0,)), ((), ())),
            preferred_element_type=jnp.float32,
            precision=jax.lax.Precision.HIGHEST,
        )
        m1 = jnp.min(sims, axis=1, keepdims=True)
        j1 = jnp.min(jnp.where(sims == m1, col, _HW), axis=1, keepdims=True)
        s1 = jnp.where(col == j1, jnp.inf, sims)
        m2 = jnp.min(s1, axis=1, keepdims=True)
        j2 = jnp.min(jnp.where(s1 == m2, col, _HW), axis=1, keepdims=True)
        s2 = jnp.where(col == j2, jnp.inf, s1)
        m3 = jnp.min(s2, axis=1, keepdims=True)
        bot3 = m1 + m2 + m3

        hinge = jnp.maximum(bot3 - sims + _MARGIN, 0.0)

        gx = gx_ref[b]
        gy = gy_ref[b]
        xmin = jnp.floor(gx * _IMG).astype(jnp.int32)
        ymin = jnp.floor(gy * _IMG).astype(jnp.int32)
        valid = (xmin >= 0) & (ymin >= 0) & (xmin + 1 <= _IMG) & (ymin + 1 <= _IMG)
        x0 = jnp.floor_divide(xmin, 8)
        x1 = jnp.floor_divide(xmin + 1, 8)
        y0 = jnp.floor_divide(ymin, 8)
        y1 = jnp.floor_divide(ymin + 1, 8)
        mx0 = (x0 >= 0) & (x0 <= _F)
        mx1 = (x1 != x0) & (x1 >= 0) & (x1 <= _F)
        my0 = (y0 >= 0) & (y0 <= _F)
        my1 = (y1 != y0) & (y1 >= 0) & (y1 <= _F)

        w = jnp.zeros((_HW, _HW), jnp.float32)
        for xs, ys, mj in ((x0, y0, mx0 & my0), (x0, y1, mx0 & my1),
                           (x1, y0, mx1 & my0), (x1, y1, mx1 & my1)):
            ia = jnp.clip(ys, 0, _F - 1) * _F + jnp.clip(xs, 0, _F - 1)
            mjv = mj & valid
            w = w + jnp.where((row == ia) & mjv, 1.0, 0.0)
            den = den + jnp.sum(mjv.astype(jnp.float32))
        num = num + jnp.sum(w * hinge)
    out_ref[...] = jnp.broadcast_to(num / (1e-6 + den), (1, 1))


def kernel(sketch_query_vectors, ref_key_vectors, G):
    B = sketch_query_vectors.shape[0]
    sq = sketch_query_vectors.reshape(B, _C, _HW)
    rk = ref_key_vectors.reshape(B, _C, _HW)
    mid = G[:, 4::8, 4::8, :]
    gx = mid[..., 0].reshape(B, 1, _HW)
    gy = mid[..., 1].reshape(B, 1, _HW)
    out = pl.pallas_call(
        _triplet_kernel,
        out_shape=jax.ShapeDtypeStruct((1, 1), jnp.float32),
    )(sq, rk, gx, gy)
    return out[0, 0]
